# baseline (speedup 1.0000x reference)
import functools

import jax
import jax.numpy as jnp
from jax import lax
from jax.experimental import pallas as pl
from jax.experimental.pallas import tpu as pltpu
from jax.experimental.pallas import tpu_sc as plsc

_LANES = 16


def _make_sc_index(B, N, T, W):
    mesh = plsc.VectorSubcoreMesh(core_axis_name="c", subcore_axis_name="s")
    num_workers = mesh.num_cores * mesh.num_subcores
    per_worker = B // num_workers

    @functools.partial(
        pl.kernel,
        out_type=jax.ShapeDtypeStruct((B // 8, 8, N), jnp.int32),
        mesh=mesh,
        scratch_types=[
            pltpu.VMEM((2, N), jnp.float32),
            pltpu.VMEM((N,), jnp.int32),
        ],
    )
    def sc_index(dw_hbm, out_hbm, dwv, cv):
        wid = lax.axis_index("s") * mesh.num_cores + lax.axis_index("c")
        for t in range(per_worker):
            b = wid * per_worker + t
            pltpu.sync_copy(dw_hbm.at[b], dwv)

            def chunk(i, carry):
                sl = pl.ds(i * _LANES, _LANES)
                d = jnp.clip(dwv[0, sl] * float(T), 0.0, float(T - 1)).astype(jnp.int32)
                w = jnp.clip(dwv[1, sl], 0.0, float(W - 1)).astype(jnp.int32)
                cv[sl] = d * 8 + w
                return carry

            lax.fori_loop(0, N // _LANES, chunk, 0)
            pltpu.sync_copy(cv, out_hbm.at[b >> 3, b & 7])

    return sc_index


def _tc_body(c_ref, td_ref, tw_ref, out_ref):
    T = td_ref.shape[0]
    W = tw_ref.shape[0]
    BB = c_ref.shape[1]
    N = c_ref.shape[2]

    iota_t = jax.lax.broadcasted_iota(jnp.int32, (T, N), 0)
    iota_w = jax.lax.broadcasted_iota(jnp.int32, (W, N), 0)
    for i in range(BB):
        c = c_ref[0, i:i + 1, :]
        d_idx = c >> 3
        w_idx = c & 7
        oh_d = (iota_t == d_idx).astype(jnp.float32)
        oh_w = (iota_w == w_idx).astype(jnp.float32)
        acc = jax.lax.dot_general(td_ref[...], oh_d, (((0,), (0,)), ((), ())),
                                  preferred_element_type=jnp.float32)
        acc = acc + jax.lax.dot_general(tw_ref[...], oh_w, (((0,), (0,)), ((), ())),
                                        preferred_element_type=jnp.float32)
        out_ref[i, :, :] = acc


def kernel(x, time_day, time_week):
    B, S, N, C = x.shape
    T, F = time_day.shape
    W = time_week.shape[0]

    dw = jnp.transpose(x[:, -1, :, 1:3], (0, 2, 1))
    cidx = _make_sc_index(B, N, T, W)(dw)

    BB = 16
    out = pl.pallas_call(
        _tc_body,
        grid=(B // BB,),
        in_specs=[
            pl.BlockSpec((1, BB, N), lambda b: (b, 0, 0)),
            pl.BlockSpec((T, F), lambda b: (0, 0)),
            pl.BlockSpec((W, F), lambda b: (0, 0)),
        ],
        out_specs=pl.BlockSpec((BB, F, N), lambda b: (b, 0, 0)),
        out_shape=jax.ShapeDtypeStruct((B, F, N), jnp.float32),
    )(cidx, time_day, time_week)
    return out[..., None]


# speedup vs baseline: 1.0157x; 1.0157x over previous
import functools

import jax
import jax.numpy as jnp
from jax import lax
from jax.experimental import pallas as pl
from jax.experimental.pallas import tpu as pltpu
from jax.experimental.pallas import tpu_sc as plsc

_LANES = 16


def _make_sc_index(B, N, T, W):
    mesh = plsc.VectorSubcoreMesh(core_axis_name="c", subcore_axis_name="s")
    num_workers = mesh.num_cores * mesh.num_subcores
    per_worker = B // num_workers

    @functools.partial(
        pl.kernel,
        out_type=jax.ShapeDtypeStruct((B // 8, 8, N), jnp.int32),
        mesh=mesh,
        scratch_types=[
            pltpu.VMEM((2, N), jnp.float32),
            pltpu.VMEM((N,), jnp.int32),
        ],
    )
    def sc_index(dw_hbm, out_hbm, dwv, cv):
        wid = lax.axis_index("s") * mesh.num_cores + lax.axis_index("c")
        for t in range(per_worker):
            b = wid * per_worker + t
            pltpu.sync_copy(dw_hbm.at[b], dwv)

            def chunk(i, carry):
                sl = pl.ds(i * _LANES, _LANES)
                d = jnp.clip(dwv[0, sl] * float(T), 0.0, float(T - 1)).astype(jnp.int32)
                w = jnp.clip(dwv[1, sl], 0.0, float(W - 1)).astype(jnp.int32)
                cv[sl] = d * 8 + w
                return carry

            lax.fori_loop(0, N // _LANES, chunk, 0)
            pltpu.sync_copy(cv, out_hbm.at[b >> 3, b & 7])

    return sc_index


def _tc_body(c_ref, td_ref, tw_ref, out_ref):
    T = td_ref.shape[0]
    W = tw_ref.shape[0]
    BB = c_ref.shape[1]
    N = c_ref.shape[2]

    iota_t = jax.lax.broadcasted_iota(jnp.int32, (T, N), 0)
    iota_w = jax.lax.broadcasted_iota(jnp.int32, (W, N), 0)
    for i in range(BB):
        c = c_ref[0, i:i + 1, :]
        d_idx = c >> 3
        w_idx = c & 7
        oh_d = (iota_t == d_idx).astype(jnp.float32)
        oh_w = (iota_w == w_idx).astype(jnp.float32)
        acc = jax.lax.dot_general(td_ref[...], oh_d, (((0,), (0,)), ((), ())),
                                  preferred_element_type=jnp.float32)
        acc = acc + jax.lax.dot_general(tw_ref[...], oh_w, (((0,), (0,)), ((), ())),
                                        preferred_element_type=jnp.float32)
        out_ref[i, :, :] = acc


def kernel(x, time_day, time_week):
    B, S, N, C = x.shape
    T, F = time_day.shape
    W = time_week.shape[0]

    dw = jnp.transpose(x[:, -1, :, 1:3], (0, 2, 1))
    cidx = _make_sc_index(B, N, T, W)(dw)

    BB = 8
    out = pl.pallas_call(
        _tc_body,
        grid=(B // BB,),
        in_specs=[
            pl.BlockSpec((1, BB, N), lambda b: (b, 0, 0)),
            pl.BlockSpec((T, F), lambda b: (0, 0)),
            pl.BlockSpec((W, F), lambda b: (0, 0)),
        ],
        out_specs=pl.BlockSpec((BB, F, N), lambda b: (b, 0, 0)),
        out_shape=jax.ShapeDtypeStruct((B, F, N), jnp.float32),
    )(cidx, time_day, time_week)
    return out[..., None]
